# baseline (speedup 1.0000x reference)
import functools

import jax
import jax.numpy as jnp
from jax import lax
from jax.experimental import pallas as pl
from jax.experimental.pallas import tpu as pltpu
from jax.experimental.pallas import tpu_sc as plsc

N_NODES = 10000
N_EDGES = 320000
D = 128
NC = 2
NS = 16
NW = NC * NS
N_PAD = 10240
EPT = N_EDGES // NW
CH = 128
SB = 8
NBLK = 10
NCHUNK = SB * NBLK
EPT_PAD = NCHUNK * CH
ROWS_PT = N_PAD // NS
ZCHUNKS = [(0, 128), (128, 128), (256, 128), (384, 128), (512, 128)]

_f32 = jnp.float32


def _deg_body(dst_hbm, deg_hbm, deg_sh, idx_v, ones_v, zbuf):
    c = lax.axis_index("c")
    s = lax.axis_index("s")
    wid = c * NS + s

    def z16(i, _):
        zbuf[pl.ds(i * 16, 16)] = jnp.zeros((16,), _f32)
        return 0

    lax.fori_loop(0, ROWS_PT // 16, z16, 0)
    pltpu.sync_copy(zbuf,
                    deg_sh.at[pl.ds(s * ROWS_PT, ROWS_PT)])

    def o16(i, _):
        ones_v[pl.ds(i * 16, 16)] = jnp.ones((16,), _f32)
        return 0

    lax.fori_loop(0, CH // 16, o16, 0)
    pltpu.sync_copy(dst_hbm.at[wid], idx_v)
    plsc.subcore_barrier()

    def chunk(j, _):
        pltpu.sync_copy(ones_v, deg_sh.at[idx_v.at[j]], add=True)
        return 0

    lax.fori_loop(0, NCHUNK, chunk, 0)
    plsc.subcore_barrier()
    pltpu.sync_copy(deg_sh.at[pl.ds(s * ROWS_PT, ROWS_PT)],
                    deg_hbm.at[c, pl.ds(s * ROWS_PT, ROWS_PT)])


def _dis_of(deg_ref):
    deg = deg_ref[0, :] + deg_ref[1, :]
    return jnp.where(deg > 0, lax.rsqrt(jnp.maximum(deg, 1e-12)), 0.0)


def _mm_body(x_ref, w_ref, xw_ref):
    xw_ref[...] = jnp.dot(x_ref[...], w_ref[...], preferred_element_type=_f32)


BR = 1024


def _scale_body(deg_ref, xw_ref, y_ref):
    i = pl.program_id(0)
    dis = _dis_of(deg_ref)
    rows = i * BR + lax.broadcasted_iota(jnp.int32, (BR, 1), 0)
    y_ref[...] = jnp.where(rows < N_NODES, xw_ref[...] * dis[:, None], 0.0)


NBUF = 2


def _edge_body(y_hbm, src_hbm, dst_hbm, part_hbm, acc_sh, srcb, dstb,
               r0, r1, si0, si1, s0, s1):
    rows = (r0, r1)
    sems = (s0, s1)
    si = (si0, si1)
    c = lax.axis_index("c")
    s = lax.axis_index("s")
    wid = c * NS + s

    def z(i, _):
        r0[i // 8, pl.ds((i % 8) * 16, 16)] = jnp.zeros((16,), _f32)
        return 0

    lax.fori_loop(0, CH * 8, z, 0)
    for off, ln in ZCHUNKS:
        pltpu.sync_copy(r0.at[pl.ds(0, ln)],
                        acc_sh.at[pl.ds(s * ROWS_PT + off, ln)])
    pltpu.sync_copy(src_hbm.at[wid, pl.ds(0, SB)], srcb.at[0])
    pltpu.sync_copy(dst_hbm.at[wid, pl.ds(0, SB)], dstb.at[0])
    plsc.subcore_barrier()

    def do_block(blk, slot):
        for b in range(NBUF - 1):
            pltpu.async_copy(y_hbm.at[srcb.at[slot, b]], rows[b], sems[b])

        def step(ii, _):
            for b in range(NBUF):
                jl = NBUF * ii + b
                cb = (b + NBUF - 1) % NBUF
                pltpu.make_async_copy(y_hbm.at[srcb.at[slot, jl]], rows[b],
                                      sems[b]).wait()
                pltpu.async_copy(rows[b], acc_sh.at[dstb.at[slot, jl]],
                                 sems[b], add=True)
                @pl.when(jl >= 1)
                def _():
                    pltpu.make_async_copy(rows[cb],
                                          acc_sh.at[dstb.at[slot, jl - 1]],
                                          sems[cb]).wait()

                @pl.when(jl + NBUF - 1 < SB)
                def _():
                    pltpu.async_copy(y_hbm.at[srcb.at[slot, jl + NBUF - 1]],
                                     rows[cb], sems[cb])
            return 0

        lax.fori_loop(0, SB // NBUF, step, 0)
        fb = (SB - 1) % NBUF
        pltpu.make_async_copy(rows[fb], acc_sh.at[dstb.at[slot, SB - 1]],
                              sems[fb]).wait()

    def pair(p, _):
        for slot in range(2):
            blk = 2 * p + slot
            nslot = 1 - slot
            @pl.when(blk >= 1)
            def _():
                pltpu.make_async_copy(src_hbm.at[wid, pl.ds(blk * SB, SB)],
                                      srcb.at[slot], si[slot]).wait()
                pltpu.make_async_copy(dst_hbm.at[wid, pl.ds(blk * SB, SB)],
                                      dstb.at[slot], si[slot]).wait()

            @pl.when(blk + 1 < NBLK)
            def _():
                pltpu.async_copy(
                    src_hbm.at[wid, pl.ds((blk + 1) * SB, SB)],
                    srcb.at[nslot], si[nslot])
                pltpu.async_copy(
                    dst_hbm.at[wid, pl.ds((blk + 1) * SB, SB)],
                    dstb.at[nslot], si[nslot])

            do_block(blk, slot)
        return 0

    lax.fori_loop(0, NBLK // 2, pair, 0)
    plsc.subcore_barrier()

    for off, ln in ZCHUNKS:
        pltpu.sync_copy(acc_sh.at[pl.ds(s * ROWS_PT + off, ln)],
                        part_hbm.at[c, pl.ds(s * ROWS_PT + off, ln)])


def _fin_body(part_ref, deg_ref, b_ref, out_ref):
    dis = _dis_of(deg_ref)
    p = part_ref[0, :, :] + part_ref[1, :, :]
    out_ref[...] = p * dis[:, None] + b_ref[...][None, :]


@functools.lru_cache(maxsize=1)
def _build():
    mesh = plsc.VectorSubcoreMesh(core_axis_name="c", subcore_axis_name="s",
                                  num_cores=NC, num_subcores=NS)
    deg_call = pl.kernel(
        _deg_body,
        out_type=jax.ShapeDtypeStruct((NC, N_PAD), _f32),
        mesh=mesh,
        scratch_types=[
            pltpu.VMEM_SHARED((N_PAD,), _f32),
            pltpu.VMEM((NCHUNK, CH), jnp.int32),
            pltpu.VMEM((CH,), _f32),
            pltpu.VMEM((ROWS_PT,), _f32),
        ],
    )
    mm_call = pl.pallas_call(
        _mm_body,
        grid=(N_NODES // 1000,),
        in_specs=[pl.BlockSpec((1000, D), lambda i: (i, 0)),
                  pl.BlockSpec((D, D), lambda i: (0, 0))],
        out_specs=pl.BlockSpec((1000, D), lambda i: (i, 0)),
        out_shape=jax.ShapeDtypeStruct((N_NODES, D), _f32),
    )
    scale_call = pl.pallas_call(
        _scale_body,
        grid=(N_PAD // BR,),
        in_specs=[pl.BlockSpec((NC, BR), lambda i: (0, i)),
                  pl.BlockSpec((BR, D), lambda i: (i, 0))],
        out_specs=pl.BlockSpec((BR, D), lambda i: (i, 0)),
        out_shape=jax.ShapeDtypeStruct((N_PAD, D), _f32),
    )
    edge_call = pl.kernel(
        _edge_body,
        out_type=jax.ShapeDtypeStruct((NC, N_PAD, D), _f32),
        mesh=mesh,
        scratch_types=[
            pltpu.VMEM_SHARED((N_PAD, D), _f32),
            pltpu.VMEM((2, SB, CH), jnp.int32),
            pltpu.VMEM((2, SB, CH), jnp.int32),
            pltpu.VMEM((CH, D), _f32),
            pltpu.VMEM((CH, D), _f32),
            pltpu.SemaphoreType.DMA,
            pltpu.SemaphoreType.DMA,
            pltpu.SemaphoreType.DMA,
            pltpu.SemaphoreType.DMA,
        ],
    )
    fin_call = pl.pallas_call(
        _fin_body,
        grid=(N_PAD // BR,),
        in_specs=[pl.BlockSpec((NC, BR, D), lambda i: (0, i, 0)),
                  pl.BlockSpec((NC, BR), lambda i: (0, i)),
                  pl.BlockSpec((D,), lambda i: (0,))],
        out_specs=pl.BlockSpec((BR, D), lambda i: (i, 0)),
        out_shape=jax.ShapeDtypeStruct((N_NODES, D), _f32),
    )
    return deg_call, mm_call, scale_call, edge_call, fin_call


def kernel(x, edge_index, W, b):
    deg_call, mm_call, scale_call, edge_call, fin_call = _build()
    src = edge_index[0].astype(jnp.int32)
    dst = edge_index[1].astype(jnp.int32)
    padi = N_NODES + (jnp.arange(EPT_PAD - EPT, dtype=jnp.int32)
                      % (N_PAD - N_NODES))
    pad_t = jnp.broadcast_to(padi, (NW, EPT_PAD - EPT))
    src_p = jnp.concatenate([src.reshape(NW, EPT), pad_t],
                            axis=1).reshape(NW, NCHUNK, CH)
    dst_p = jnp.concatenate([dst.reshape(NW, EPT), pad_t],
                            axis=1).reshape(NW, NCHUNK, CH)
    deg_p = deg_call(dst_p)
    xw = mm_call(x, W)
    y = scale_call(deg_p, xw)
    part = edge_call(y, src_p, dst_p)
    out = fin_call(part, deg_p, b)
    return (out, edge_index)


# speedup vs baseline: 1.2363x; 1.2363x over previous
import functools

import jax
import jax.numpy as jnp
from jax import lax
from jax.experimental import pallas as pl
from jax.experimental.pallas import tpu as pltpu
from jax.experimental.pallas import tpu_sc as plsc

N_NODES = 10000
N_EDGES = 320000
D = 128
NC = 2
NS = 16
NW = NC * NS
N_PAD = 10240
EPT = N_EDGES // NW
CH = 64
SB = 10
NBLK = 16
NCHUNK = SB * NBLK
EPT_PAD = NCHUNK * CH
CH_D = 128
NCHUNK_D = 80
ROWS_PT = N_PAD // NS
ZCHUNKS = [(k * 64, 64) for k in range(10)]

_f32 = jnp.float32


def _deg_body(dst_hbm, deg_hbm, deg_sh, idx_v, ones_v, zbuf):
    c = lax.axis_index("c")
    s = lax.axis_index("s")
    wid = c * NS + s

    def z16(i, _):
        zbuf[pl.ds(i * 16, 16)] = jnp.zeros((16,), _f32)
        return 0

    lax.fori_loop(0, ROWS_PT // 16, z16, 0)
    pltpu.sync_copy(zbuf,
                    deg_sh.at[pl.ds(s * ROWS_PT, ROWS_PT)])

    def o16(i, _):
        ones_v[pl.ds(i * 16, 16)] = jnp.ones((16,), _f32)
        return 0

    lax.fori_loop(0, CH_D // 16, o16, 0)
    pltpu.sync_copy(dst_hbm.at[wid], idx_v)
    plsc.subcore_barrier()

    def chunk(j, _):
        pltpu.sync_copy(ones_v, deg_sh.at[idx_v.at[j]], add=True)
        return 0

    lax.fori_loop(0, NCHUNK_D, chunk, 0)
    plsc.subcore_barrier()
    pltpu.sync_copy(deg_sh.at[pl.ds(s * ROWS_PT, ROWS_PT)],
                    deg_hbm.at[c, pl.ds(s * ROWS_PT, ROWS_PT)])


def _dis_of(deg_ref):
    deg = deg_ref[0, :] + deg_ref[1, :]
    return jnp.where(deg > 0, lax.rsqrt(jnp.maximum(deg, 1e-12)), 0.0)


def _mm_body(x_ref, w_ref, xw_ref):
    xw_ref[...] = jnp.dot(x_ref[...], w_ref[...], preferred_element_type=_f32)


BR = 2048


def _scale_body(deg_ref, xw_ref, y_ref):
    i = pl.program_id(0)
    dis = _dis_of(deg_ref)
    rows = i * BR + lax.broadcasted_iota(jnp.int32, (BR, 1), 0)
    y_ref[...] = jnp.where(rows < N_NODES, xw_ref[...] * dis[:, None], 0.0)


NBUF = 5


def _edge_body(y_hbm, src_hbm, dst_hbm, part_hbm, acc_sh, srcb, dstb,
               r0, r1, r2, r3, r4, si0, si1, s0, s1, s2, s3, s4):
    rows = (r0, r1, r2, r3, r4)
    sems = (s0, s1, s2, s3, s4)
    si = (si0, si1)
    c = lax.axis_index("c")
    s = lax.axis_index("s")
    wid = c * NS + s

    def z(i, _):
        r0[i // 8, pl.ds((i % 8) * 16, 16)] = jnp.zeros((16,), _f32)
        return 0

    lax.fori_loop(0, CH * 8, z, 0)
    for off, ln in ZCHUNKS:
        pltpu.sync_copy(r0.at[pl.ds(0, ln)],
                        acc_sh.at[pl.ds(s * ROWS_PT + off, ln)])
    pltpu.sync_copy(src_hbm.at[wid, 0], srcb.at[0])
    pltpu.sync_copy(dst_hbm.at[wid, 0], dstb.at[0])
    plsc.subcore_barrier()
    for b in range(NBUF - 1):
        pltpu.async_copy(y_hbm.at[srcb.at[0, b]], rows[b], sems[b])

    FB = (SB - 1) % NBUF

    def gwait(slot, jl, b):
        pltpu.make_async_copy(y_hbm.at[srcb.at[slot, jl]], rows[b],
                              sems[b]).wait()

    def sissue(slot, jl, b):
        pltpu.async_copy(rows[b], acc_sh.at[dstb.at[slot, jl]], sems[b],
                         add=True)

    def swait(slot, jl, b):
        pltpu.make_async_copy(rows[b], acc_sh.at[dstb.at[slot, jl]],
                              sems[b]).wait()

    def gissue(slot, jl, b):
        pltpu.async_copy(y_hbm.at[srcb.at[slot, jl]], rows[b], sems[b])

    def do_block(blk, slot, nslot):
        def step(ii, _):
            for b in range(NBUF):
                jl = NBUF * ii + b
                cb = (b + NBUF - 1) % NBUF
                gwait(slot, jl, b)
                sissue(slot, jl, b)
                @pl.when(jl >= 1)
                def _():
                    swait(slot, jl - 1, cb)

                @pl.when(jl + NBUF - 1 < SB)
                def _():
                    gissue(slot, jl + NBUF - 1, cb)
            return 0

        lax.fori_loop(0, SB // NBUF - 1, step, 0)
        @pl.when(blk + 1 < NBLK)
        def _():
            pltpu.make_async_copy(src_hbm.at[wid, blk + 1], srcb.at[nslot],
                                  si[nslot]).wait()
            pltpu.make_async_copy(dst_hbm.at[wid, blk + 1], dstb.at[nslot],
                                  si[nslot]).wait()

        for b in range(NBUF):
            jl = SB - NBUF + b
            cb = (b + NBUF - 1) % NBUF
            gwait(slot, jl, b)
            sissue(slot, jl, b)
            swait(slot, jl - 1, cb)
            t = jl + NBUF - 1
            if t < SB:
                gissue(slot, t, cb)
            else:
                @pl.when(blk + 1 < NBLK)
                def _():
                    gissue(nslot, t - SB, cb)

    def pair(p, _):
        for slot in range(2):
            blk = 2 * p + slot
            nslot = 1 - slot
            @pl.when(blk >= 1)
            def _():
                swait(nslot, SB - 1, FB)

            @pl.when(blk + 1 < NBLK)
            def _():
                pltpu.async_copy(src_hbm.at[wid, blk + 1], srcb.at[nslot],
                                 si[nslot])
                pltpu.async_copy(dst_hbm.at[wid, blk + 1], dstb.at[nslot],
                                 si[nslot])

            do_block(blk, slot, nslot)
        return 0

    lax.fori_loop(0, NBLK // 2, pair, 0)
    swait(1, SB - 1, FB)
    plsc.subcore_barrier()

    for off, ln in ZCHUNKS:
        pltpu.sync_copy(acc_sh.at[pl.ds(s * ROWS_PT + off, ln)],
                        part_hbm.at[c, pl.ds(s * ROWS_PT + off, ln)])


def _fin_body(part_ref, deg_ref, b_ref, out_ref):
    dis = _dis_of(deg_ref)
    p = part_ref[0, :, :] + part_ref[1, :, :]
    out_ref[...] = p * dis[:, None] + b_ref[...][None, :]


@functools.lru_cache(maxsize=1)
def _build():
    mesh = plsc.VectorSubcoreMesh(core_axis_name="c", subcore_axis_name="s",
                                  num_cores=NC, num_subcores=NS)
    deg_call = pl.kernel(
        _deg_body,
        out_type=jax.ShapeDtypeStruct((NC, N_PAD), _f32),
        mesh=mesh,
        scratch_types=[
            pltpu.VMEM_SHARED((N_PAD,), _f32),
            pltpu.VMEM((NCHUNK_D, CH_D), jnp.int32),
            pltpu.VMEM((CH_D,), _f32),
            pltpu.VMEM((ROWS_PT,), _f32),
        ],
    )
    mm_call = pl.pallas_call(
        _mm_body,
        grid=(N_NODES // 1000,),
        in_specs=[pl.BlockSpec((1000, D), lambda i: (i, 0)),
                  pl.BlockSpec((D, D), lambda i: (0, 0))],
        out_specs=pl.BlockSpec((1000, D), lambda i: (i, 0)),
        out_shape=jax.ShapeDtypeStruct((N_NODES, D), _f32),
    )
    scale_call = pl.pallas_call(
        _scale_body,
        grid=(N_PAD // BR,),
        in_specs=[pl.BlockSpec((NC, BR), lambda i: (0, i)),
                  pl.BlockSpec((BR, D), lambda i: (i, 0))],
        out_specs=pl.BlockSpec((BR, D), lambda i: (i, 0)),
        out_shape=jax.ShapeDtypeStruct((N_PAD, D), _f32),
    )
    edge_call = pl.kernel(
        _edge_body,
        out_type=jax.ShapeDtypeStruct((NC, N_PAD, D), _f32),
        mesh=mesh,
        scratch_types=[
            pltpu.VMEM_SHARED((N_PAD, D), _f32),
            pltpu.VMEM((2, SB, CH), jnp.int32),
            pltpu.VMEM((2, SB, CH), jnp.int32),
            pltpu.VMEM((CH, D), _f32),
            pltpu.VMEM((CH, D), _f32),
            pltpu.VMEM((CH, D), _f32),
            pltpu.VMEM((CH, D), _f32),
            pltpu.VMEM((CH, D), _f32),
            pltpu.SemaphoreType.DMA,
            pltpu.SemaphoreType.DMA,
            pltpu.SemaphoreType.DMA,
            pltpu.SemaphoreType.DMA,
            pltpu.SemaphoreType.DMA,
            pltpu.SemaphoreType.DMA,
            pltpu.SemaphoreType.DMA,
        ],
    )
    fin_call = pl.pallas_call(
        _fin_body,
        grid=(N_PAD // BR,),
        in_specs=[pl.BlockSpec((NC, BR, D), lambda i: (0, i, 0)),
                  pl.BlockSpec((NC, BR), lambda i: (0, i)),
                  pl.BlockSpec((D,), lambda i: (0,))],
        out_specs=pl.BlockSpec((BR, D), lambda i: (i, 0)),
        out_shape=jax.ShapeDtypeStruct((N_NODES, D), _f32),
    )
    return deg_call, mm_call, scale_call, edge_call, fin_call


def kernel(x, edge_index, W, b):
    deg_call, mm_call, scale_call, edge_call, fin_call = _build()
    src = edge_index[0].astype(jnp.int32)
    dst = edge_index[1].astype(jnp.int32)
    padi = N_NODES + (jnp.arange(EPT_PAD - EPT, dtype=jnp.int32)
                      % (N_PAD - N_NODES))
    pad_t = jnp.broadcast_to(padi, (NW, EPT_PAD - EPT))
    src_p = jnp.concatenate([src.reshape(NW, EPT), pad_t],
                            axis=1).reshape(NW, NBLK, SB, CH)
    dst_p = jnp.concatenate([dst.reshape(NW, EPT), pad_t],
                            axis=1).reshape(NW, NBLK, SB, CH)
    dst_d = jnp.concatenate([dst.reshape(NW, EPT), pad_t],
                            axis=1).reshape(NW, NCHUNK_D, CH_D)
    deg_p = deg_call(dst_d)
    xw = mm_call(x, W)
    y = scale_call(deg_p, xw)
    part = edge_call(y, src_p, dst_p)
    out = fin_call(part, deg_p, b)
    return (out, edge_index)
